# Initial kernel scaffold; baseline (speedup 1.0000x reference)
import jax
import jax.numpy as jnp
from jax.experimental import pallas as pl


def kernel(source, W):
    raise NotImplementedError("write your pallas kernel here")


# speedup vs baseline: 4.4803x; 4.4803x over previous
import functools

import numpy as np
import jax
import jax.numpy as jnp
from jax import lax
from jax.experimental import pallas as pl
from jax.experimental.pallas import tpu as pltpu
from jax.experimental.pallas import tpu_sc as plsc

_NC = 2
_NS = 16
_NW = _NC * _NS
_LANES = 16
_CHUNK = 128


def _pe_rows(seq_len: int, dim: int) -> np.ndarray:
    pe = np.zeros((seq_len, dim), dtype=np.float32)
    position = np.arange(0, seq_len, dtype=np.float32)[:, None]
    div_term = np.exp(
        np.arange(0, dim, 2, dtype=np.float32) * -(np.log(10000.0) / dim)
    )
    pe[:, 0::2] = np.sin(position * div_term)
    pe[:, 1::2] = np.cos(position * div_term)
    return pe


@functools.lru_cache(maxsize=None)
def _build(seq_len: int, batch: int, vocab: int, dim: int):
    n_rows = seq_len * batch
    assert n_rows % (_NW * _CHUNK) == 0
    assert batch % _CHUNK == 0 and dim % _LANES == 0
    rows_per_w = n_rows // _NW
    n_chunks = rows_per_w // _CHUNK
    lane_groups = dim // _LANES
    scale = float(np.sqrt(float(dim)))
    log2_batch = int(np.log2(batch))
    assert (1 << log2_batch) == batch

    mesh = plsc.VectorSubcoreMesh(
        core_axis_name="c", subcore_axis_name="s",
        num_cores=_NC, num_subcores=_NS,
    )

    @functools.partial(
        pl.kernel,
        out_type=jax.ShapeDtypeStruct((n_rows, dim), jnp.float32),
        mesh=mesh,
        scratch_types=[
            pltpu.VMEM((rows_per_w,), jnp.int32),
            pltpu.VMEM((seq_len * dim,), jnp.float32),
            pltpu.VMEM((_CHUNK, dim), jnp.float32),
            pltpu.SemaphoreType.DMA,
        ],
    )
    def launch(words_hbm, pe_hbm, w_hbm, out_hbm, idx_v, pe_v, buf_v, sem):
        wid = lax.axis_index("s") * _NC + lax.axis_index("c")
        base = wid * rows_per_w
        pltpu.sync_copy(words_hbm.at[pl.ds(base, rows_per_w)], idx_v)
        pltpu.sync_copy(pe_hbm, pe_v)

        def chunk_body(k, carry):
            row0 = base + k * _CHUNK
            seq = row0 >> log2_batch
            pltpu.async_copy(
                w_hbm.at[idx_v.at[pl.ds(k * _CHUNK, _CHUNK)]], buf_v, sem
            ).wait()
            pe_off = seq * dim
            pe_regs = [
                pe_v[pl.ds(pe_off + l * _LANES, _LANES)]
                for l in range(lane_groups)
            ]

            def row_body(j, c):
                for l in range(lane_groups):
                    v = buf_v[j, pl.ds(l * _LANES, _LANES)]
                    buf_v[j, pl.ds(l * _LANES, _LANES)] = v * scale + pe_regs[l]
                return c

            lax.fori_loop(0, _CHUNK, row_body, 0)
            pltpu.sync_copy(buf_v, out_hbm.at[pl.ds(row0, _CHUNK)])
            return carry

        lax.fori_loop(0, n_chunks, chunk_body, 0)

    return launch


def kernel(source, W):
    seq_len, batch, _ = source.shape
    vocab, dim = W.shape
    words = source.reshape(seq_len * batch)
    pe = jnp.asarray(_pe_rows(seq_len, dim).reshape(-1))
    launch = _build(seq_len, batch, vocab, dim)
    out = launch(words, pe, W)
    return out.reshape(seq_len, batch, dim)


# speedup vs baseline: 7.6885x; 1.7161x over previous
import functools

import numpy as np
import jax
import jax.numpy as jnp
from jax import lax
from jax.experimental import pallas as pl
from jax.experimental.pallas import tpu as pltpu
from jax.experimental.pallas import tpu_sc as plsc

_NC = 2
_NS = 16
_NW = _NC * _NS
_LANES = 16
_CHUNK = 128


def _pe_rows(seq_len: int, dim: int) -> np.ndarray:
    pe = np.zeros((seq_len, dim), dtype=np.float32)
    position = np.arange(0, seq_len, dtype=np.float32)[:, None]
    div_term = np.exp(
        np.arange(0, dim, 2, dtype=np.float32) * -(np.log(10000.0) / dim)
    )
    pe[:, 0::2] = np.sin(position * div_term)
    pe[:, 1::2] = np.cos(position * div_term)
    return pe


@functools.lru_cache(maxsize=None)
def _build(seq_len: int, batch: int, vocab: int, dim: int):
    n_rows = seq_len * batch
    assert n_rows % (_NW * _CHUNK) == 0
    assert batch % _CHUNK == 0 and dim % _LANES == 0
    rows_per_w = n_rows // _NW
    n_chunks = rows_per_w // _CHUNK
    lane_groups = dim // _LANES
    scale = float(np.sqrt(float(dim)))
    log2_batch = int(np.log2(batch))
    assert (1 << log2_batch) == batch

    mesh = plsc.VectorSubcoreMesh(
        core_axis_name="c", subcore_axis_name="s",
        num_cores=_NC, num_subcores=_NS,
    )

    assert n_chunks % 2 == 0

    @functools.partial(
        pl.kernel,
        out_type=jax.ShapeDtypeStruct((n_rows, dim), jnp.float32),
        mesh=mesh,
        scratch_types=[
            pltpu.VMEM((rows_per_w,), jnp.int32),
            pltpu.VMEM((seq_len * dim,), jnp.float32),
            pltpu.VMEM((_CHUNK, dim), jnp.float32),
            pltpu.VMEM((_CHUNK, dim), jnp.float32),
            pltpu.VMEM((_CHUNK, dim), jnp.float32),
            pltpu.VMEM((_CHUNK, dim), jnp.float32),
            pltpu.SemaphoreType.DMA,
            pltpu.SemaphoreType.DMA,
            pltpu.SemaphoreType.DMA,
            pltpu.SemaphoreType.DMA,
        ],
    )
    def launch(words_hbm, pe_hbm, w_hbm, out_hbm, idx_v, pe_v,
               buf_a, buf_b, buf_c, buf_d, in_a, in_b, out_c, out_d):
        wid = lax.axis_index("s") * _NC + lax.axis_index("c")
        base = wid * rows_per_w
        pltpu.sync_copy(words_hbm.at[pl.ds(base, rows_per_w)], idx_v)
        pltpu.sync_copy(pe_hbm, pe_v)

        def gather(k, buf, sem):
            return pltpu.make_async_copy(
                w_hbm.at[idx_v.at[pl.ds(k * _CHUNK, _CHUNK)]], buf, sem
            )

        def store(k, buf, sem):
            return pltpu.make_async_copy(
                buf, out_hbm.at[pl.ds(base + k * _CHUNK, _CHUNK)], sem
            )

        def compute(k, src, dst):
            seq = (base + k * _CHUNK) >> log2_batch
            pe_off = seq * dim
            pe_regs = [
                pe_v[pl.ds(pe_off + l * _LANES, _LANES)]
                for l in range(lane_groups)
            ]

            def row_body(j, c):
                for l in range(lane_groups):
                    v = src[j, pl.ds(l * _LANES, _LANES)]
                    dst[j, pl.ds(l * _LANES, _LANES)] = v * scale + pe_regs[l]
                return c

            lax.fori_loop(0, _CHUNK, row_body, 0)

        gather(0, buf_a, in_a).start()
        gather(1, buf_b, in_b).start()

        @pl.loop(0, n_chunks, step=2)
        def pipelined(k):
            for (kk, g_buf, g_sem, r_buf, r_sem) in (
                (k, buf_a, in_a, buf_c, out_c),
                (k + 1, buf_b, in_b, buf_d, out_d),
            ):
                @pl.when(kk >= 2)
                def _():
                    store(kk - 2, r_buf, r_sem).wait()

                gather(kk, g_buf, g_sem).wait()
                compute(kk, g_buf, r_buf)
                store(kk, r_buf, r_sem).start()

                @pl.when(kk + 2 < n_chunks)
                def _():
                    gather(kk + 2, g_buf, g_sem).start()

        store(n_chunks - 2, buf_c, out_c).wait()
        store(n_chunks - 1, buf_d, out_d).wait()

    return launch


def kernel(source, W):
    seq_len, batch, _ = source.shape
    vocab, dim = W.shape
    words = source.reshape(seq_len * batch)
    pe = jnp.asarray(_pe_rows(seq_len, dim).reshape(-1))
    launch = _build(seq_len, batch, vocab, dim)
    out = launch(words, pe, W)
    return out.reshape(seq_len, batch, dim)
